# baseline (speedup 1.0000x reference)
import jax
import jax.numpy as jnp
from jax.experimental import pallas as pl
from jax.experimental.pallas import tpu as pltpu
from jax.experimental.pallas import tpu_sc as plsc

D = 64
H = 32
NW = 32


def _sc_gather_pairs(table_wide, gidx, chunk):
    n = gidx.shape[0]
    w = table_wide.shape[1]
    chunks = n // (NW * chunk)
    mesh = plsc.VectorSubcoreMesh(core_axis_name="core", subcore_axis_name="subcore")

    @pl.kernel(
        out_type=jax.ShapeDtypeStruct((n, w), table_wide.dtype),
        mesh=mesh,
        scratch_types=[
            pltpu.VMEM((chunk,), jnp.int32),
            pltpu.VMEM((chunk, w), table_wide.dtype),
            pltpu.SemaphoreType.DMA,
        ],
    )
    def kern(tab_hbm, i_hbm, o_hbm, idx_v, rows_v, sem):
        wid = jax.lax.axis_index("subcore") * 2 + jax.lax.axis_index("core")

        @pl.loop(0, chunks)
        def _(c):
            base = (wid * chunks + c) * chunk
            pltpu.sync_copy(i_hbm.at[pl.ds(base, chunk)], idx_v)
            pltpu.async_copy(tab_hbm.at[idx_v], rows_v, sem).wait()
            pltpu.sync_copy(rows_v, o_hbm.at[pl.ds(base, chunk)])

    return kern(table_wide, gidx)


def _tc_select(wide, par, blk):
    n = wide.shape[0]

    def body(w_ref, p_ref, o_ref):
        w = w_ref[...]
        o_ref[...] = jnp.where(p_ref[...] == 1, w[:, D:], w[:, :D])

    return pl.pallas_call(
        body,
        grid=(n // blk,),
        in_specs=[
            pl.BlockSpec((blk, 2 * D), lambda i: (i, 0)),
            pl.BlockSpec((blk, 1), lambda i: (i, 0)),
        ],
        out_specs=pl.BlockSpec((blk, D), lambda i: (i, 0)),
        out_shape=jax.ShapeDtypeStruct((n, D), jnp.float32),
    )(wide, par)


def _tc_select_mlp(wide, par, w1t, b1, w2t, b2, blk):
    n = wide.shape[0]

    def body(w_ref, p_ref, w1_ref, b1_ref, w2_ref, b2_ref, o_ref):
        w = w_ref[...]
        x = jnp.where(p_ref[...] == 1, w[:, D:], w[:, :D])
        h = jnp.dot(x, w1_ref[...], preferred_element_type=jnp.float32)
        h = jnp.maximum(h + b1_ref[...], 0.0)
        o_ref[...] = (
            jnp.dot(h, w2_ref[...], preferred_element_type=jnp.float32) + b2_ref[...]
        )

    return pl.pallas_call(
        body,
        grid=(n // blk,),
        in_specs=[
            pl.BlockSpec((blk, 2 * D), lambda i: (i, 0)),
            pl.BlockSpec((blk, 1), lambda i: (i, 0)),
            pl.BlockSpec((D, H), lambda i: (0, 0)),
            pl.BlockSpec((1, H), lambda i: (0, 0)),
            pl.BlockSpec((H, D), lambda i: (0, 0)),
            pl.BlockSpec((1, D), lambda i: (0, 0)),
        ],
        out_specs=pl.BlockSpec((blk, D), lambda i: (i, 0)),
        out_shape=jax.ShapeDtypeStruct((n, D), jnp.float32),
    )(wide, par, w1t, b1.reshape(1, H), w2t, b2.reshape(1, D))


def kernel(x, E_main, E_compl, W1, b1, W2, b2):
    bsz, f = x.shape
    v = E_main.shape[0]
    idx_main = x[:, 0]
    idx_compl = x[:, 1:].reshape(-1)

    main_wide = _sc_gather_pairs(E_main.reshape(v // 2, 2 * D), idx_main >> 1, 512)
    out_main = _tc_select_mlp(
        main_wide, (idx_main & 1).reshape(-1, 1), W1.T, b1, W2.T, b2, 2048
    )

    compl_wide = _sc_gather_pairs(E_compl.reshape(v // 2, 2 * D), idx_compl >> 1, 512)
    x_compl = _tc_select(compl_wide, (idx_compl & 1).reshape(-1, 1), 2048)
    return (out_main, x_compl.reshape(bsz, f - 1, D))
